# Initial kernel scaffold; baseline (speedup 1.0000x reference)
import jax
import jax.numpy as jnp
from jax.experimental import pallas as pl


def kernel(input_data, table):
    raise NotImplementedError("write your pallas kernel here")


# speedup vs baseline: 1.1031x; 1.1031x over previous
import functools

import jax
import jax.numpy as jnp
from jax import lax
from jax.experimental import pallas as pl
from jax.experimental.pallas import tpu as pltpu
from jax.experimental.pallas import tpu_sc as plsc

VOCAB = 1000000
DIM = 32
TOTAL = 16384 * 50
NW = 32
PER_W = TOTAL // NW
CHUNK = 1600
NCHUNK = PER_W // CHUNK

_mesh = plsc.VectorSubcoreMesh(core_axis_name="c", subcore_axis_name="s")


@functools.partial(
    pl.kernel,
    mesh=_mesh,
    compiler_params=pltpu.CompilerParams(use_tc_tiling_on_sc=False),
    out_type=jax.ShapeDtypeStruct((TOTAL, DIM), jnp.float32),
    scratch_types=[
        pltpu.VMEM((CHUNK,), jnp.int32),
        pltpu.VMEM((CHUNK, DIM), jnp.float32),
        pltpu.SemaphoreType.DMA,
    ],
)
def _gather_kernel(idx_hbm, table_hbm, out_hbm, idx_v, rows_v, sem):
    wid = lax.axis_index("s") * 2 + lax.axis_index("c")
    base = wid * PER_W

    def body(i, carry):
        off = base + i * CHUNK
        pltpu.sync_copy(idx_hbm.at[pl.ds(off, CHUNK)], idx_v)
        pltpu.async_copy(table_hbm.at[idx_v], rows_v, sem).wait()
        pltpu.sync_copy(rows_v, out_hbm.at[pl.ds(off, CHUNK)])
        return carry

    lax.fori_loop(0, NCHUNK, body, 0)


def kernel(input_data, table):
    idx = input_data.reshape(TOTAL).astype(jnp.int32)
    out = _gather_kernel(idx, table)
    return out.reshape(input_data.shape + (DIM,))


# speedup vs baseline: 1.1095x; 1.0057x over previous
import functools

import jax
import jax.numpy as jnp
from jax import lax
from jax.experimental import pallas as pl
from jax.experimental.pallas import tpu as pltpu
from jax.experimental.pallas import tpu_sc as plsc

VOCAB = 1000000
DIM = 32
TOTAL = 16384 * 50
NW = 32
PER_W = TOTAL // NW
CHUNK = 1600
NCHUNK = PER_W // CHUNK

_mesh = plsc.VectorSubcoreMesh(core_axis_name="c", subcore_axis_name="s")


@functools.partial(
    pl.kernel,
    mesh=_mesh,
    compiler_params=pltpu.CompilerParams(use_tc_tiling_on_sc=False),
    out_type=jax.ShapeDtypeStruct((TOTAL, DIM), jnp.float32),
    scratch_types=[
        pltpu.VMEM((CHUNK,), jnp.int32),
        pltpu.VMEM((CHUNK,), jnp.int32),
        pltpu.VMEM((CHUNK, DIM), jnp.float32),
        pltpu.VMEM((CHUNK, DIM), jnp.float32),
        pltpu.SemaphoreType.DMA,
        pltpu.SemaphoreType.DMA,
        pltpu.SemaphoreType.DMA,
        pltpu.SemaphoreType.DMA,
        pltpu.SemaphoreType.DMA,
        pltpu.SemaphoreType.DMA,
    ],
)
def _gather_kernel(idx_hbm, table_hbm, out_hbm, idx_v0, idx_v1, rows_v0,
                   rows_v1, is0, is1, gs0, gs1, os0, os1):
    wid = lax.axis_index("s") * 2 + lax.axis_index("c")
    base = wid * PER_W

    idx_v = (idx_v0, idx_v1)
    rows_v = (rows_v0, rows_v1)
    isem = (is0, is1)
    gsem = (gs0, gs1)
    osem = (os0, os1)

    def load_idx(i):
        return pltpu.async_copy(
            idx_hbm.at[pl.ds(base + i * CHUNK, CHUNK)], idx_v[i % 2],
            isem[i % 2])

    def gather(i):
        return pltpu.async_copy(table_hbm.at[idx_v[i % 2]], rows_v[i % 2],
                                gsem[i % 2])

    def writeback(i):
        return pltpu.async_copy(rows_v[i % 2],
                                out_hbm.at[pl.ds(base + i * CHUNK, CHUNK)],
                                osem[i % 2])

    il = [load_idx(0), load_idx(1)]
    il[0].wait()
    g = [gather(0), None]
    ow = [None, None]

    for i in range(NCHUNK):
        b = i % 2
        nb = 1 - b
        g[b].wait()
        if i + 1 < NCHUNK:
            il[nb].wait()
            if ow[nb] is not None:
                ow[nb].wait()
            g[nb] = gather(i + 1)
        ow[b] = writeback(i)
        if i + 2 < NCHUNK:
            il[b] = load_idx(i + 2)
    ow[0].wait()
    ow[1].wait()


def kernel(input_data, table):
    idx = input_data.reshape(TOTAL).astype(jnp.int32)
    out = _gather_kernel(idx, table)
    return out.reshape(input_data.shape + (DIM,))


# speedup vs baseline: 1.7902x; 1.6136x over previous
import functools

import jax
import jax.numpy as jnp
from jax import lax
from jax.experimental import pallas as pl
from jax.experimental.pallas import tpu as pltpu
from jax.experimental.pallas import tpu_sc as plsc

VOCAB = 1000000
DIM = 32
TOTAL = 16384 * 50
NW = 32
PER_W = TOTAL // NW
CHUNK = 1600
NCHUNK = PER_W // CHUNK

_mesh = plsc.VectorSubcoreMesh(core_axis_name="c", subcore_axis_name="s")


@functools.partial(
    pl.kernel,
    mesh=_mesh,
    compiler_params=pltpu.CompilerParams(use_tc_tiling_on_sc=False),
    out_type=jax.ShapeDtypeStruct((16384, 50, DIM), jnp.float32),
    scratch_types=[
        pltpu.VMEM((CHUNK,), jnp.int32),
        pltpu.VMEM((CHUNK,), jnp.int32),
        pltpu.VMEM((CHUNK, DIM), jnp.float32),
        pltpu.VMEM((CHUNK, DIM), jnp.float32),
        pltpu.SemaphoreType.DMA,
        pltpu.SemaphoreType.DMA,
        pltpu.SemaphoreType.DMA,
        pltpu.SemaphoreType.DMA,
        pltpu.SemaphoreType.DMA,
        pltpu.SemaphoreType.DMA,
    ],
)
def _gather_kernel(idx_hbm, table_hbm, out_hbm, idx_v0, idx_v1, rows_v0,
                   rows_v1, is0, is1, gs0, gs1, os0, os1):
    wid = lax.axis_index("s") * 2 + lax.axis_index("c")
    base = wid * PER_W

    idx_v = (idx_v0, idx_v1)
    rows_v = (rows_v0, rows_v1)
    isem = (is0, is1)
    gsem = (gs0, gs1)
    osem = (os0, os1)

    def load_idx(i):
        return pltpu.async_copy(
            idx_hbm.at[pl.ds(base + i * CHUNK, CHUNK)], idx_v[i % 2],
            isem[i % 2])

    def gather(i):
        return pltpu.async_copy(table_hbm.at[idx_v[i % 2]], rows_v[i % 2],
                                gsem[i % 2])

    def writeback(i):
        row0 = (base + i * CHUNK) // 50
        return [
            pltpu.async_copy(rows_v[i % 2].at[pl.ds(r * 50, 50)],
                             out_hbm.at[row0 + r], osem[i % 2])
            for r in range(CHUNK // 50)
        ]

    il = [load_idx(0), load_idx(1)]
    il[0].wait()
    g = [gather(0), None]
    ow = [None, None]

    for i in range(NCHUNK):
        b = i % 2
        nb = 1 - b
        g[b].wait()
        if i + 1 < NCHUNK:
            il[nb].wait()
            if ow[nb] is not None:
                for cp in ow[nb]:
                    cp.wait()
            g[nb] = gather(i + 1)
        ow[b] = writeback(i)
        if i + 2 < NCHUNK:
            il[b] = load_idx(i + 2)
    for cp in ow[0]:
        cp.wait()
    for cp in ow[1]:
        cp.wait()


def kernel(input_data, table):
    idx = input_data.reshape(TOTAL).astype(jnp.int32)
    return _gather_kernel(idx, table)
